# baseline (speedup 1.0000x reference)
import functools

import jax
import jax.numpy as jnp
from jax import lax
from jax.experimental import pallas as pl
from jax.experimental.pallas import tpu as pltpu
from jax.experimental.pallas import tpu_sc as plsc

N = 10000
E = 320000
D = 128
H = 128
OUT = 64
G = 64

NC = 2
NS = 16
NW = NC * NS
C = 104
NBUF = 2
CHUNKS_PER_TILE = 97
EPT = CHUNKS_PER_TILE * C
GRP = CHUNKS_PER_TILE
NAGG = N + 8
ZTILES = 10
ZROWS = N // ZTILES

_mesh = plsc.VectorSubcoreMesh(core_axis_name="c", subcore_axis_name="s")


@functools.partial(
    pl.kernel,
    out_type=jax.ShapeDtypeStruct((NC, N, H), jnp.float32),
    mesh=_mesh,
    scratch_types=[
        pltpu.VMEM((EPT,), jnp.int32),
        pltpu.VMEM((CHUNKS_PER_TILE, C), jnp.int32),
        pltpu.VMEM((NBUF, C, H), jnp.float32),
        pltpu.VMEM_SHARED((NAGG, H), jnp.float32),
        pltpu.SemaphoreType.DMA((NBUF,)),
    ],
)
def _sc_gather_scatter(src_hbm, dst_hbm, h_hbm, zeros_hbm, out_hbm,
                       src_v, dst_v, rows_v, agg_sh, gsems):
    c = lax.axis_index("c")
    s = lax.axis_index("s")
    wid = c * NS + s

    pltpu.sync_copy(src_hbm.at[pl.ds(wid * EPT, EPT)], src_v)
    pltpu.sync_copy(dst_hbm.at[wid], dst_v)

    @pl.when(s < ZTILES)
    def _():
        pltpu.sync_copy(zeros_hbm, agg_sh.at[pl.ds(s * ZROWS, ZROWS)])

    @pl.when(s == ZTILES)
    def _():
        pltpu.sync_copy(zeros_hbm.at[pl.ds(0, NAGG - N)],
                        agg_sh.at[pl.ds(N, NAGG - N)])

    plsc.subcore_barrier()

    def issue(k, j):
        return pltpu.async_copy(
            h_hbm.at[src_v.at[pl.ds(k * C, C)]], rows_v.at[j], gsems.at[j])

    def body(g, _):
        base = g * GRP
        copies = {0: issue(base, 0), 1: issue(base + 1, 1)}
        for i in range(GRP):
            j = i % NBUF
            copies[i].wait()
            pltpu.sync_copy(rows_v.at[j], agg_sh.at[dst_v.at[base + i]],
                            add=True)
            if i + NBUF < GRP:
                copies[i + NBUF] = issue(base + i + NBUF, j)
        return 0

    lax.fori_loop(0, CHUNKS_PER_TILE // GRP, body, 0)

    plsc.subcore_barrier()

    @pl.when(s < ZTILES)
    def _():
        pltpu.sync_copy(agg_sh.at[pl.ds(s * ZROWS, ZROWS)],
                        out_hbm.at[c, pl.ds(s * ZROWS, ZROWS)])


RB = 2000
NB = N // RB


def _mlp_body(h_ref, agg_ref, batch_ref, w1_ref, b1_ref, w2_ref, b2_ref,
              h_out_ref, pooled_ref):
    i = pl.program_id(0)
    z = h_ref[...] + agg_ref[0] + agg_ref[1]
    t = jnp.maximum(
        jnp.dot(z, w1_ref[...], preferred_element_type=jnp.float32)
        + b1_ref[...], 0.0)
    h2 = jnp.maximum(
        jnp.dot(t, w2_ref[...], preferred_element_type=jnp.float32)
        + b2_ref[...], 0.0)
    h_out_ref[...] = h2
    bblk = batch_ref[0, 0, :]
    onehot = (bblk[:, None] ==
              lax.broadcasted_iota(jnp.int32, (RB, G), 1)).astype(jnp.float32)
    contrib = lax.dot_general(onehot, h2, (((0,), (0,)), ((), ())),
                              preferred_element_type=jnp.float32)

    @pl.when(i == 0)
    def _():
        pooled_ref[...] = jnp.zeros_like(pooled_ref)

    pooled_ref[...] += contrib


_mlp_call = pl.pallas_call(
    _mlp_body,
    grid=(NB,),
    in_specs=[
        pl.BlockSpec((RB, H), lambda i: (i, 0)),
        pl.BlockSpec((NC, RB, H), lambda i: (0, i, 0)),
        pl.BlockSpec((1, 1, RB), lambda i: (i, 0, 0)),
        pl.BlockSpec((H, H), lambda i: (0, 0)),
        pl.BlockSpec((1, H), lambda i: (0, 0)),
        pl.BlockSpec((H, H), lambda i: (0, 0)),
        pl.BlockSpec((1, H), lambda i: (0, 0)),
    ],
    out_specs=[
        pl.BlockSpec((RB, H), lambda i: (i, 0)),
        pl.BlockSpec((G, H), lambda i: (0, 0)),
    ],
    out_shape=[
        jax.ShapeDtypeStruct((N, H), jnp.float32),
        jax.ShapeDtypeStruct((G, H), jnp.float32),
    ],
)


def _ffn_body(p0_ref, p1_ref, p2_ref, wf1_ref, bf1_ref, wf2_ref, bf2_ref,
              out_ref):
    t = (jnp.dot(p0_ref[...], wf1_ref[0], preferred_element_type=jnp.float32)
         + jnp.dot(p1_ref[...], wf1_ref[1], preferred_element_type=jnp.float32)
         + jnp.dot(p2_ref[...], wf1_ref[2], preferred_element_type=jnp.float32)
         + bf1_ref[...])
    t = jnp.maximum(t, 0.0)
    out_ref[...] = (jnp.dot(t, wf2_ref[...],
                            preferred_element_type=jnp.float32)
                    + bf2_ref[...])


_ffn_call = pl.pallas_call(
    _ffn_body,
    out_shape=jax.ShapeDtypeStruct((G, OUT), jnp.float32),
)


@jax.jit
def kernel(x, edge_index, batch,
           W1_0, b1_0, W2_0, b2_0,
           W1_1, b1_1, W2_1, b2_1,
           W1_2, b1_2, W2_2, b2_2,
           Wf1, bf1, Wf2, bf2):
    src2 = jnp.concatenate(
        [edge_index[0].reshape(NW, E // NW),
         jnp.zeros((NW, EPT - E // NW), jnp.int32)], axis=1).reshape(-1)
    dst2 = jnp.concatenate(
        [edge_index[1].reshape(NW, E // NW),
         jnp.full((NW, EPT - E // NW), N, jnp.int32)],
        axis=1).reshape(NW, CHUNKS_PER_TILE, C)
    zeros = jnp.zeros((ZROWS, H), jnp.float32)
    batch3 = batch.reshape(NB, 1, RB)

    h = x.astype(jnp.float32)
    layers = [(W1_0, b1_0, W2_0, b2_0),
              (W1_1, b1_1, W2_1, b2_1),
              (W1_2, b1_2, W2_2, b2_2)]
    pooled = []
    for (W1, b1, W2, b2) in layers:
        agg = _sc_gather_scatter(src2, dst2, h, zeros)
        h, p = _mlp_call(h, agg, batch3,
                         W1, b1.reshape(1, H), W2, b2.reshape(1, H))
        pooled.append(p)

    return _ffn_call(pooled[0], pooled[1], pooled[2],
                     Wf1.reshape(3, H, H // 2), bf1.reshape(1, H // 2),
                     Wf2, bf2.reshape(1, OUT))


# speedup vs baseline: 1.0482x; 1.0482x over previous
import functools

import jax
import jax.numpy as jnp
from jax import lax
from jax.experimental import pallas as pl
from jax.experimental.pallas import tpu as pltpu
from jax.experimental.pallas import tpu_sc as plsc

N = 10000
E = 320000
D = 128
H = 128
OUT = 64
G = 64

NC = 2
NS = 16
NW = NC * NS
C = 112
NBUF = 2
CHUNKS_PER_TILE = 90
EPT = CHUNKS_PER_TILE * C
GRP = CHUNKS_PER_TILE
NAGG = N
NPAD = NW * (EPT - E // NW)
ZTILES = 10
ZROWS = N // ZTILES

_mesh = plsc.VectorSubcoreMesh(core_axis_name="c", subcore_axis_name="s")


@functools.partial(
    pl.kernel,
    out_type=jax.ShapeDtypeStruct((NC, N, H), jnp.float32),
    mesh=_mesh,
    scratch_types=[
        pltpu.VMEM((EPT,), jnp.int32),
        pltpu.VMEM((CHUNKS_PER_TILE, C), jnp.int32),
        pltpu.VMEM((NBUF, C, H), jnp.float32),
        pltpu.VMEM_SHARED((NAGG, H), jnp.float32),
        pltpu.SemaphoreType.DMA((NBUF,)),
    ],
)
def _sc_gather_scatter(src_hbm, dst_hbm, h_hbm, zeros_hbm, out_hbm,
                       src_v, dst_v, rows_v, agg_sh, gsems):
    c = lax.axis_index("c")
    s = lax.axis_index("s")
    wid = c * NS + s

    pltpu.sync_copy(src_hbm.at[pl.ds(wid * EPT, EPT)], src_v)
    pltpu.sync_copy(dst_hbm.at[wid], dst_v)

    @pl.when(s < ZTILES)
    def _():
        pltpu.sync_copy(zeros_hbm, agg_sh.at[pl.ds(s * ZROWS, ZROWS)])

    plsc.subcore_barrier()

    def issue(k, j):
        return pltpu.async_copy(
            h_hbm.at[src_v.at[pl.ds(k * C, C)]], rows_v.at[j], gsems.at[j])

    def body(g, _):
        base = g * GRP
        copies = {0: issue(base, 0), 1: issue(base + 1, 1)}
        for i in range(GRP):
            j = i % NBUF
            copies[i].wait()
            pltpu.sync_copy(rows_v.at[j], agg_sh.at[dst_v.at[base + i]],
                            add=True)
            if i + NBUF < GRP:
                copies[i + NBUF] = issue(base + i + NBUF, j)
        return 0

    lax.fori_loop(0, CHUNKS_PER_TILE // GRP, body, 0)

    plsc.subcore_barrier()

    @pl.when(s < ZTILES)
    def _():
        pltpu.sync_copy(agg_sh.at[pl.ds(s * ZROWS, ZROWS)],
                        out_hbm.at[c, pl.ds(s * ZROWS, ZROWS)])


RB = 2000
NB = N // RB


def _mlp_body(h_ref, agg_ref, batch_ref, w1_ref, b1_ref, w2_ref, b2_ref,
              h_out_ref, pooled_ref):
    i = pl.program_id(0)
    z = h_ref[...] + agg_ref[0] + agg_ref[1]
    factor = jnp.where(i == 0, jnp.float32(NPAD), jnp.float32(0.0))
    rowmask = (lax.broadcasted_iota(jnp.int32, (RB, 1), 0) == 0
               ).astype(jnp.float32)
    z = z - (factor * rowmask) * h_ref[0:1, :]
    t = jnp.maximum(
        jnp.dot(z, w1_ref[...], preferred_element_type=jnp.float32)
        + b1_ref[...], 0.0)
    h2 = jnp.maximum(
        jnp.dot(t, w2_ref[...], preferred_element_type=jnp.float32)
        + b2_ref[...], 0.0)
    h_out_ref[...] = h2
    bblk = batch_ref[0, 0, :]
    onehot = (bblk[:, None] ==
              lax.broadcasted_iota(jnp.int32, (RB, G), 1)).astype(jnp.float32)
    contrib = lax.dot_general(onehot, h2, (((0,), (0,)), ((), ())),
                              preferred_element_type=jnp.float32)

    @pl.when(i == 0)
    def _():
        pooled_ref[...] = jnp.zeros_like(pooled_ref)

    pooled_ref[...] += contrib


_mlp_call = pl.pallas_call(
    _mlp_body,
    grid=(NB,),
    in_specs=[
        pl.BlockSpec((RB, H), lambda i: (i, 0)),
        pl.BlockSpec((NC, RB, H), lambda i: (0, i, 0)),
        pl.BlockSpec((1, 1, RB), lambda i: (i, 0, 0)),
        pl.BlockSpec((H, H), lambda i: (0, 0)),
        pl.BlockSpec((1, H), lambda i: (0, 0)),
        pl.BlockSpec((H, H), lambda i: (0, 0)),
        pl.BlockSpec((1, H), lambda i: (0, 0)),
    ],
    out_specs=[
        pl.BlockSpec((RB, H), lambda i: (i, 0)),
        pl.BlockSpec((G, H), lambda i: (0, 0)),
    ],
    out_shape=[
        jax.ShapeDtypeStruct((N, H), jnp.float32),
        jax.ShapeDtypeStruct((G, H), jnp.float32),
    ],
)


def _ffn_body(p0_ref, p1_ref, p2_ref, wf1_ref, bf1_ref, wf2_ref, bf2_ref,
              out_ref):
    t = (jnp.dot(p0_ref[...], wf1_ref[0], preferred_element_type=jnp.float32)
         + jnp.dot(p1_ref[...], wf1_ref[1], preferred_element_type=jnp.float32)
         + jnp.dot(p2_ref[...], wf1_ref[2], preferred_element_type=jnp.float32)
         + bf1_ref[...])
    t = jnp.maximum(t, 0.0)
    out_ref[...] = (jnp.dot(t, wf2_ref[...],
                            preferred_element_type=jnp.float32)
                    + bf2_ref[...])


_ffn_call = pl.pallas_call(
    _ffn_body,
    out_shape=jax.ShapeDtypeStruct((G, OUT), jnp.float32),
)


@jax.jit
def kernel(x, edge_index, batch,
           W1_0, b1_0, W2_0, b2_0,
           W1_1, b1_1, W2_1, b2_1,
           W1_2, b1_2, W2_2, b2_2,
           Wf1, bf1, Wf2, bf2):
    src2 = jnp.concatenate(
        [edge_index[0].reshape(NW, E // NW),
         jnp.zeros((NW, EPT - E // NW), jnp.int32)], axis=1).reshape(-1)
    dst2 = jnp.concatenate(
        [edge_index[1].reshape(NW, E // NW),
         jnp.zeros((NW, EPT - E // NW), jnp.int32)],
        axis=1).reshape(NW, CHUNKS_PER_TILE, C)
    zeros = jnp.zeros((ZROWS, H), jnp.float32)
    batch3 = batch.reshape(NB, 1, RB)

    h = x.astype(jnp.float32)
    layers = [(W1_0, b1_0, W2_0, b2_0),
              (W1_1, b1_1, W2_1, b2_1),
              (W1_2, b1_2, W2_2, b2_2)]
    pooled = []
    for (W1, b1, W2, b2) in layers:
        agg = _sc_gather_scatter(src2, dst2, h, zeros)
        h, p = _mlp_call(h, agg, batch3,
                         W1, b1.reshape(1, H), W2, b2.reshape(1, H))
        pooled.append(p)

    return _ffn_call(pooled[0], pooled[1], pooled[2],
                     Wf1.reshape(3, H, H // 2), bf1.reshape(1, H // 2),
                     Wf2, bf2.reshape(1, OUT))


# speedup vs baseline: 1.6742x; 1.5973x over previous
import functools

import jax
import jax.numpy as jnp
from jax import lax
from jax.experimental import pallas as pl
from jax.experimental.pallas import tpu as pltpu
from jax.experimental.pallas import tpu_sc as plsc

N = 10000
E = 320000
D = 128
H = 128
OUT = 64
G = 64

NC = 2
NS = 16
NW = NC * NS
C = 80
NBUF = 2
CHUNKS_PER_TILE = E // (NW * C)
EPT = CHUNKS_PER_TILE * C
GRP = CHUNKS_PER_TILE
NAGG = N
ZTILES = 10
ZROWS = N // ZTILES

_mesh = plsc.VectorSubcoreMesh(core_axis_name="c", subcore_axis_name="s")


@functools.partial(
    pl.kernel,
    out_type=jax.ShapeDtypeStruct((NC, N, H), jnp.float32),
    mesh=_mesh,
    scratch_types=[
        pltpu.VMEM((EPT,), jnp.int32),
        pltpu.VMEM((CHUNKS_PER_TILE, C), jnp.int32),
        pltpu.VMEM((NBUF, C, H), jnp.float32),
        pltpu.VMEM_SHARED((NAGG, H), jnp.float32),
        pltpu.SemaphoreType.DMA((NBUF,)),
    ],
)
def _sc_gather_scatter(src_hbm, dst_hbm, h_hbm, zeros_hbm, out_hbm,
                       src_v, dst_v, rows_v, agg_sh, gsems):
    c = lax.axis_index("c")
    s = lax.axis_index("s")
    wid = c * NS + s

    pltpu.sync_copy(src_hbm.at[pl.ds(wid * EPT, EPT)], src_v)
    pltpu.sync_copy(dst_hbm.at[wid], dst_v)

    @pl.when(s < ZTILES)
    def _():
        pltpu.sync_copy(zeros_hbm, agg_sh.at[pl.ds(s * ZROWS, ZROWS)])

    plsc.subcore_barrier()

    def issue(k, j):
        return pltpu.async_copy(
            h_hbm.at[src_v.at[pl.ds(k * C, C)]], rows_v.at[j], gsems.at[j])

    def body(g, _):
        base = g * GRP
        copies = {0: issue(base, 0), 1: issue(base + 1, 1)}
        for i in range(GRP):
            j = i % NBUF
            copies[i].wait()
            pltpu.sync_copy(rows_v.at[j], agg_sh.at[dst_v.at[base + i]],
                            add=True)
            if i + NBUF < GRP:
                copies[i + NBUF] = issue(base + i + NBUF, j)
        return 0

    lax.fori_loop(0, CHUNKS_PER_TILE // GRP, body, 0)

    plsc.subcore_barrier()

    @pl.when(s < ZTILES)
    def _():
        pltpu.sync_copy(agg_sh.at[pl.ds(s * ZROWS, ZROWS)],
                        out_hbm.at[c, pl.ds(s * ZROWS, ZROWS)])


RB = 2000
NB = N // RB


def _mlp_body(h_ref, agg_ref, batch_ref, w1_ref, b1_ref, w2_ref, b2_ref,
              h_out_ref, pooled_ref):
    i = pl.program_id(0)
    z = h_ref[...] + agg_ref[0] + agg_ref[1]
    t = jnp.maximum(
        jnp.dot(z, w1_ref[...], preferred_element_type=jnp.float32)
        + b1_ref[...], 0.0)
    h2 = jnp.maximum(
        jnp.dot(t, w2_ref[...], preferred_element_type=jnp.float32)
        + b2_ref[...], 0.0)
    h_out_ref[...] = h2
    bblk = batch_ref[0, 0, :]
    onehot = (bblk[:, None] ==
              lax.broadcasted_iota(jnp.int32, (RB, G), 1)).astype(jnp.float32)
    contrib = lax.dot_general(onehot, h2, (((0,), (0,)), ((), ())),
                              preferred_element_type=jnp.float32)

    @pl.when(i == 0)
    def _():
        pooled_ref[...] = jnp.zeros_like(pooled_ref)

    pooled_ref[...] += contrib


_mlp_call = pl.pallas_call(
    _mlp_body,
    grid=(NB,),
    in_specs=[
        pl.BlockSpec((RB, H), lambda i: (i, 0)),
        pl.BlockSpec((NC, RB, H), lambda i: (0, i, 0)),
        pl.BlockSpec((1, 1, RB), lambda i: (i, 0, 0)),
        pl.BlockSpec((H, H), lambda i: (0, 0)),
        pl.BlockSpec((1, H), lambda i: (0, 0)),
        pl.BlockSpec((H, H), lambda i: (0, 0)),
        pl.BlockSpec((1, H), lambda i: (0, 0)),
    ],
    out_specs=[
        pl.BlockSpec((RB, H), lambda i: (i, 0)),
        pl.BlockSpec((G, H), lambda i: (0, 0)),
    ],
    out_shape=[
        jax.ShapeDtypeStruct((N, H), jnp.float32),
        jax.ShapeDtypeStruct((G, H), jnp.float32),
    ],
)


def _ffn_body(p0_ref, p1_ref, p2_ref, wf1_ref, bf1_ref, wf2_ref, bf2_ref,
              out_ref):
    t = (jnp.dot(p0_ref[...], wf1_ref[0], preferred_element_type=jnp.float32)
         + jnp.dot(p1_ref[...], wf1_ref[1], preferred_element_type=jnp.float32)
         + jnp.dot(p2_ref[...], wf1_ref[2], preferred_element_type=jnp.float32)
         + bf1_ref[...])
    t = jnp.maximum(t, 0.0)
    out_ref[...] = (jnp.dot(t, wf2_ref[...],
                            preferred_element_type=jnp.float32)
                    + bf2_ref[...])


_ffn_call = pl.pallas_call(
    _ffn_body,
    out_shape=jax.ShapeDtypeStruct((G, OUT), jnp.float32),
)


@jax.jit
def kernel(x, edge_index, batch,
           W1_0, b1_0, W2_0, b2_0,
           W1_1, b1_1, W2_1, b2_1,
           W1_2, b1_2, W2_2, b2_2,
           Wf1, bf1, Wf2, bf2):
    src2 = edge_index[0]
    dst2 = edge_index[1].reshape(NW, CHUNKS_PER_TILE, C)
    zeros = jnp.zeros((ZROWS, H), jnp.float32)
    batch3 = batch.reshape(NB, 1, RB)

    h = x.astype(jnp.float32)
    layers = [(W1_0, b1_0, W2_0, b2_0),
              (W1_1, b1_1, W2_1, b2_1),
              (W1_2, b1_2, W2_2, b2_2)]
    pooled = []
    for (W1, b1, W2, b2) in layers:
        agg = _sc_gather_scatter(src2, dst2, h, zeros)
        h, p = _mlp_call(h, agg, batch3,
                         W1, b1.reshape(1, H), W2, b2.reshape(1, H))
        pooled.append(p)

    return _ffn_call(pooled[0], pooled[1], pooled[2],
                     Wf1.reshape(3, H, H // 2), bf1.reshape(1, H // 2),
                     Wf2, bf2.reshape(1, OUT))


# speedup vs baseline: 1.6849x; 1.0064x over previous
import functools

import jax
import jax.numpy as jnp
from jax import lax
from jax.experimental import pallas as pl
from jax.experimental.pallas import tpu as pltpu
from jax.experimental.pallas import tpu_sc as plsc

N = 10000
E = 320000
D = 128
H = 128
OUT = 64
G = 64

NC = 2
NS = 16
NW = NC * NS
C = 80
NBUF = 2
CHUNKS_PER_TILE = E // (NW * C)
EPT = CHUNKS_PER_TILE * C
GRP = CHUNKS_PER_TILE
NAGG = N
ZTILES = 10
ZROWS = N // ZTILES

_mesh = plsc.VectorSubcoreMesh(core_axis_name="c", subcore_axis_name="s")


@functools.partial(
    pl.kernel,
    out_type=jax.ShapeDtypeStruct((NC, N, H), jnp.float32),
    mesh=_mesh,
    scratch_types=[
        pltpu.VMEM((EPT,), jnp.int32),
        pltpu.VMEM((CHUNKS_PER_TILE, C), jnp.int32),
        pltpu.VMEM((NBUF, C, H), jnp.float32),
        pltpu.VMEM_SHARED((NAGG, H), jnp.float32),
        pltpu.SemaphoreType.DMA((NBUF,)),
    ],
)
def _sc_gather_scatter(src_hbm, dst_hbm, h_hbm, zeros_hbm, out_hbm,
                       src_v, dst_v, rows_v, agg_sh, gsems):
    c = lax.axis_index("c")
    s = lax.axis_index("s")
    wid = c * NS + s

    pltpu.sync_copy(src_hbm.at[pl.ds(wid * EPT, EPT)], src_v)
    pltpu.sync_copy(dst_hbm.at[wid], dst_v)

    @pl.when(s < ZTILES)
    def _():
        pltpu.sync_copy(zeros_hbm, agg_sh.at[pl.ds(s * ZROWS, ZROWS)])

    plsc.subcore_barrier()

    def issue(k, j):
        return pltpu.async_copy(
            h_hbm.at[src_v.at[pl.ds(k * C, C)]], rows_v.at[j], gsems.at[j])

    def body(g, _):
        base = g * GRP
        copies = {0: issue(base, 0), 1: issue(base + 1, 1)}
        for i in range(GRP):
            j = i % NBUF
            copies[i].wait()
            pltpu.sync_copy(rows_v.at[j], agg_sh.at[dst_v.at[base + i]],
                            add=True)
            if i + NBUF < GRP:
                copies[i + NBUF] = issue(base + i + NBUF, j)
        return 0

    lax.fori_loop(0, CHUNKS_PER_TILE // GRP, body, 0)

    plsc.subcore_barrier()

    @pl.when(s < ZTILES)
    def _():
        pltpu.sync_copy(agg_sh.at[pl.ds(s * ZROWS, ZROWS)],
                        out_hbm.at[c, pl.ds(s * ZROWS, ZROWS)])


RB = 2000
NB = N // RB


def _mlp_body(h_ref, agg_ref, batch_ref, w1_ref, b1_ref, w2_ref, b2_ref,
              h_out_ref, pooled_ref):
    i = pl.program_id(0)
    z = h_ref[...] + agg_ref[0] + agg_ref[1]
    t = jnp.maximum(
        jnp.dot(z, w1_ref[...], preferred_element_type=jnp.float32)
        + b1_ref[...], 0.0)
    h2 = jnp.maximum(
        jnp.dot(t, w2_ref[...], preferred_element_type=jnp.float32)
        + b2_ref[...], 0.0)
    h_out_ref[...] = h2
    bblk = batch_ref[0, 0, :]
    onehot = (bblk[:, None] ==
              lax.broadcasted_iota(jnp.int32, (RB, G), 1)).astype(jnp.float32)
    contrib = lax.dot_general(onehot, h2, (((0,), (0,)), ((), ())),
                              preferred_element_type=jnp.float32)

    @pl.when(i == 0)
    def _():
        pooled_ref[...] = jnp.zeros_like(pooled_ref)

    pooled_ref[...] += contrib


_mlp_call = pl.pallas_call(
    _mlp_body,
    grid=(NB,),
    in_specs=[
        pl.BlockSpec((RB, H), lambda i: (i, 0)),
        pl.BlockSpec((NC, RB, H), lambda i: (0, i, 0)),
        pl.BlockSpec((1, 1, RB), lambda i: (i, 0, 0)),
        pl.BlockSpec((H, H), lambda i: (0, 0)),
        pl.BlockSpec((1, H), lambda i: (0, 0)),
        pl.BlockSpec((H, H), lambda i: (0, 0)),
        pl.BlockSpec((1, H), lambda i: (0, 0)),
    ],
    out_specs=[
        pl.BlockSpec((RB, H), lambda i: (i, 0)),
        pl.BlockSpec((G, H), lambda i: (0, 0)),
    ],
    out_shape=[
        jax.ShapeDtypeStruct((N, H), jnp.float32),
        jax.ShapeDtypeStruct((G, H), jnp.float32),
    ],
)


def _mlp_last_body(h_ref, agg_ref, batch_ref, w1_ref, b1_ref, w2_ref, b2_ref,
                   p0_ref, p1_ref, wf1_ref, bf1_ref, wf2_ref, bf2_ref,
                   out_ref, pooled_scr):
    i = pl.program_id(0)
    z = h_ref[...] + agg_ref[0] + agg_ref[1]
    t = jnp.maximum(
        jnp.dot(z, w1_ref[...], preferred_element_type=jnp.float32)
        + b1_ref[...], 0.0)
    h2 = jnp.maximum(
        jnp.dot(t, w2_ref[...], preferred_element_type=jnp.float32)
        + b2_ref[...], 0.0)
    bblk = batch_ref[0, 0, :]
    onehot = (bblk[:, None] ==
              lax.broadcasted_iota(jnp.int32, (RB, G), 1)).astype(jnp.float32)
    contrib = lax.dot_general(onehot, h2, (((0,), (0,)), ((), ())),
                              preferred_element_type=jnp.float32)

    @pl.when(i == 0)
    def _():
        pooled_scr[...] = jnp.zeros_like(pooled_scr)

    pooled_scr[...] += contrib

    @pl.when(i == NB - 1)
    def _():
        f = (jnp.dot(p0_ref[...], wf1_ref[0],
                     preferred_element_type=jnp.float32)
             + jnp.dot(p1_ref[...], wf1_ref[1],
                       preferred_element_type=jnp.float32)
             + jnp.dot(pooled_scr[...], wf1_ref[2],
                       preferred_element_type=jnp.float32)
             + bf1_ref[...])
        f = jnp.maximum(f, 0.0)
        out_ref[...] = (jnp.dot(f, wf2_ref[...],
                                preferred_element_type=jnp.float32)
                        + bf2_ref[...])


_mlp_last_call = pl.pallas_call(
    _mlp_last_body,
    grid=(NB,),
    in_specs=[
        pl.BlockSpec((RB, H), lambda i: (i, 0)),
        pl.BlockSpec((NC, RB, H), lambda i: (0, i, 0)),
        pl.BlockSpec((1, 1, RB), lambda i: (i, 0, 0)),
        pl.BlockSpec((H, H), lambda i: (0, 0)),
        pl.BlockSpec((1, H), lambda i: (0, 0)),
        pl.BlockSpec((H, H), lambda i: (0, 0)),
        pl.BlockSpec((1, H), lambda i: (0, 0)),
        pl.BlockSpec((G, H), lambda i: (0, 0)),
        pl.BlockSpec((G, H), lambda i: (0, 0)),
        pl.BlockSpec((3, H, H // 2), lambda i: (0, 0, 0)),
        pl.BlockSpec((1, H // 2), lambda i: (0, 0)),
        pl.BlockSpec((H // 2, OUT), lambda i: (0, 0)),
        pl.BlockSpec((1, OUT), lambda i: (0, 0)),
    ],
    out_specs=pl.BlockSpec((G, OUT), lambda i: (0, 0)),
    out_shape=jax.ShapeDtypeStruct((G, OUT), jnp.float32),
    scratch_shapes=[pltpu.VMEM((G, H), jnp.float32)],
)


@jax.jit
def kernel(x, edge_index, batch,
           W1_0, b1_0, W2_0, b2_0,
           W1_1, b1_1, W2_1, b2_1,
           W1_2, b1_2, W2_2, b2_2,
           Wf1, bf1, Wf2, bf2):
    src2 = edge_index[0]
    dst2 = edge_index[1].reshape(NW, CHUNKS_PER_TILE, C)
    zeros = jnp.zeros((ZROWS, H), jnp.float32)
    batch3 = batch.reshape(NB, 1, RB)

    h = x.astype(jnp.float32)
    pooled = []
    for (W1, b1, W2, b2) in [(W1_0, b1_0, W2_0, b2_0),
                             (W1_1, b1_1, W2_1, b2_1)]:
        agg = _sc_gather_scatter(src2, dst2, h, zeros)
        h, p = _mlp_call(h, agg, batch3,
                         W1, b1.reshape(1, H), W2, b2.reshape(1, H))
        pooled.append(p)

    agg = _sc_gather_scatter(src2, dst2, h, zeros)
    return _mlp_last_call(h, agg, batch3,
                          W1_2, b1_2.reshape(1, H), W2_2, b2_2.reshape(1, H),
                          pooled[0], pooled[1],
                          Wf1.reshape(3, H, H // 2), bf1.reshape(1, H // 2),
                          Wf2, bf2.reshape(1, OUT))


# speedup vs baseline: 1.6943x; 1.0056x over previous
import functools

import jax
import jax.numpy as jnp
from jax import lax
from jax.experimental import pallas as pl
from jax.experimental.pallas import tpu as pltpu
from jax.experimental.pallas import tpu_sc as plsc

N = 10000
E = 320000
D = 128
H = 128
OUT = 64
G = 64

NC = 2
NS = 16
NW = NC * NS
C = 80
NBUF = 2
CHUNKS_PER_TILE = E // (NW * C)
EPT = CHUNKS_PER_TILE * C
GRP = CHUNKS_PER_TILE
NAGG = N
ZTILES = 10
ZROWS = N // ZTILES

_mesh = plsc.VectorSubcoreMesh(core_axis_name="c", subcore_axis_name="s")


@functools.partial(
    pl.kernel,
    out_type=jax.ShapeDtypeStruct((NC, N, H), jnp.float32),
    mesh=_mesh,
    scratch_types=[
        pltpu.VMEM((EPT,), jnp.int32),
        pltpu.VMEM((CHUNKS_PER_TILE, C), jnp.int32),
        pltpu.VMEM((NBUF, C, H), jnp.float32),
        pltpu.VMEM_SHARED((NAGG, H), jnp.float32),
        pltpu.SemaphoreType.DMA((NBUF,)),
    ],
)
def _sc_gather_scatter(src_hbm, dst_hbm, h_hbm, zeros_hbm, out_hbm,
                       src_v, dst_v, rows_v, agg_sh, gsems):
    c = lax.axis_index("c")
    s = lax.axis_index("s")
    wid = c * NS + s

    pltpu.sync_copy(src_hbm.at[pl.ds(wid * EPT, EPT)], src_v)

    def issue(k, j):
        return pltpu.async_copy(
            h_hbm.at[src_v.at[pl.ds(k * C, C)]], rows_v.at[j], gsems.at[j])

    copies = {0: issue(0, 0), 1: issue(1, 1)}
    pltpu.sync_copy(dst_hbm.at[wid], dst_v)

    @pl.when(s < ZTILES)
    def _():
        pltpu.sync_copy(zeros_hbm, agg_sh.at[pl.ds(s * ZROWS, ZROWS)])

    plsc.subcore_barrier()

    for i in range(CHUNKS_PER_TILE):
        j = i % NBUF
        copies[i].wait()
        pltpu.sync_copy(rows_v.at[j], agg_sh.at[dst_v.at[i]], add=True)
        if i + NBUF < CHUNKS_PER_TILE:
            copies[i + NBUF] = issue(i + NBUF, j)

    plsc.subcore_barrier()

    @pl.when(s < ZTILES)
    def _():
        pltpu.sync_copy(agg_sh.at[pl.ds(s * ZROWS, ZROWS)],
                        out_hbm.at[c, pl.ds(s * ZROWS, ZROWS)])


RB = 2000
NB = N // RB


def _mlp_body(h_ref, agg_ref, batch_ref, w1_ref, b1_ref, w2_ref, b2_ref,
              h_out_ref, pooled_ref):
    i = pl.program_id(0)
    z = h_ref[...] + agg_ref[0] + agg_ref[1]
    t = jnp.maximum(
        jnp.dot(z, w1_ref[...], preferred_element_type=jnp.float32)
        + b1_ref[...], 0.0)
    h2 = jnp.maximum(
        jnp.dot(t, w2_ref[...], preferred_element_type=jnp.float32)
        + b2_ref[...], 0.0)
    h_out_ref[...] = h2
    bblk = batch_ref[0, 0, :]
    onehot = (bblk[:, None] ==
              lax.broadcasted_iota(jnp.int32, (RB, G), 1)).astype(jnp.float32)
    contrib = lax.dot_general(onehot, h2, (((0,), (0,)), ((), ())),
                              preferred_element_type=jnp.float32)

    @pl.when(i == 0)
    def _():
        pooled_ref[...] = jnp.zeros_like(pooled_ref)

    pooled_ref[...] += contrib


_mlp_call = pl.pallas_call(
    _mlp_body,
    grid=(NB,),
    in_specs=[
        pl.BlockSpec((RB, H), lambda i: (i, 0)),
        pl.BlockSpec((NC, RB, H), lambda i: (0, i, 0)),
        pl.BlockSpec((1, 1, RB), lambda i: (i, 0, 0)),
        pl.BlockSpec((H, H), lambda i: (0, 0)),
        pl.BlockSpec((1, H), lambda i: (0, 0)),
        pl.BlockSpec((H, H), lambda i: (0, 0)),
        pl.BlockSpec((1, H), lambda i: (0, 0)),
    ],
    out_specs=[
        pl.BlockSpec((RB, H), lambda i: (i, 0)),
        pl.BlockSpec((G, H), lambda i: (0, 0)),
    ],
    out_shape=[
        jax.ShapeDtypeStruct((N, H), jnp.float32),
        jax.ShapeDtypeStruct((G, H), jnp.float32),
    ],
)


def _mlp_last_body(h_ref, agg_ref, batch_ref, w1_ref, b1_ref, w2_ref, b2_ref,
                   p0_ref, p1_ref, wf1_ref, bf1_ref, wf2_ref, bf2_ref,
                   out_ref, pooled_scr):
    i = pl.program_id(0)
    z = h_ref[...] + agg_ref[0] + agg_ref[1]
    t = jnp.maximum(
        jnp.dot(z, w1_ref[...], preferred_element_type=jnp.float32)
        + b1_ref[...], 0.0)
    h2 = jnp.maximum(
        jnp.dot(t, w2_ref[...], preferred_element_type=jnp.float32)
        + b2_ref[...], 0.0)
    bblk = batch_ref[0, 0, :]
    onehot = (bblk[:, None] ==
              lax.broadcasted_iota(jnp.int32, (RB, G), 1)).astype(jnp.float32)
    contrib = lax.dot_general(onehot, h2, (((0,), (0,)), ((), ())),
                              preferred_element_type=jnp.float32)

    @pl.when(i == 0)
    def _():
        pooled_scr[...] = jnp.zeros_like(pooled_scr)

    pooled_scr[...] += contrib

    @pl.when(i == NB - 1)
    def _():
        f = (jnp.dot(p0_ref[...], wf1_ref[0],
                     preferred_element_type=jnp.float32)
             + jnp.dot(p1_ref[...], wf1_ref[1],
                       preferred_element_type=jnp.float32)
             + jnp.dot(pooled_scr[...], wf1_ref[2],
                       preferred_element_type=jnp.float32)
             + bf1_ref[...])
        f = jnp.maximum(f, 0.0)
        out_ref[...] = (jnp.dot(f, wf2_ref[...],
                                preferred_element_type=jnp.float32)
                        + bf2_ref[...])


_mlp_last_call = pl.pallas_call(
    _mlp_last_body,
    grid=(NB,),
    in_specs=[
        pl.BlockSpec((RB, H), lambda i: (i, 0)),
        pl.BlockSpec((NC, RB, H), lambda i: (0, i, 0)),
        pl.BlockSpec((1, 1, RB), lambda i: (i, 0, 0)),
        pl.BlockSpec((H, H), lambda i: (0, 0)),
        pl.BlockSpec((1, H), lambda i: (0, 0)),
        pl.BlockSpec((H, H), lambda i: (0, 0)),
        pl.BlockSpec((1, H), lambda i: (0, 0)),
        pl.BlockSpec((G, H), lambda i: (0, 0)),
        pl.BlockSpec((G, H), lambda i: (0, 0)),
        pl.BlockSpec((3, H, H // 2), lambda i: (0, 0, 0)),
        pl.BlockSpec((1, H // 2), lambda i: (0, 0)),
        pl.BlockSpec((H // 2, OUT), lambda i: (0, 0)),
        pl.BlockSpec((1, OUT), lambda i: (0, 0)),
    ],
    out_specs=pl.BlockSpec((G, OUT), lambda i: (0, 0)),
    out_shape=jax.ShapeDtypeStruct((G, OUT), jnp.float32),
    scratch_shapes=[pltpu.VMEM((G, H), jnp.float32)],
)


@jax.jit
def kernel(x, edge_index, batch,
           W1_0, b1_0, W2_0, b2_0,
           W1_1, b1_1, W2_1, b2_1,
           W1_2, b1_2, W2_2, b2_2,
           Wf1, bf1, Wf2, bf2):
    src2 = edge_index[0]
    dst2 = edge_index[1].reshape(NW, CHUNKS_PER_TILE, C)
    zeros = jnp.zeros((ZROWS, H), jnp.float32)
    batch3 = batch.reshape(NB, 1, RB)

    h = x.astype(jnp.float32)
    pooled = []
    for (W1, b1, W2, b2) in [(W1_0, b1_0, W2_0, b2_0),
                             (W1_1, b1_1, W2_1, b2_1)]:
        agg = _sc_gather_scatter(src2, dst2, h, zeros)
        h, p = _mlp_call(h, agg, batch3,
                         W1, b1.reshape(1, H), W2, b2.reshape(1, H))
        pooled.append(p)

    agg = _sc_gather_scatter(src2, dst2, h, zeros)
    return _mlp_last_call(h, agg, batch3,
                          W1_2, b1_2.reshape(1, H), W2_2, b2_2.reshape(1, H),
                          pooled[0], pooled[1],
                          Wf1.reshape(3, H, H // 2), bf1.reshape(1, H // 2),
                          Wf2, bf2.reshape(1, OUT))
